# baseline (speedup 1.0000x reference)
import functools

import jax
import jax.numpy as jnp
import numpy as np
from jax import lax
from jax.experimental import pallas as pl
from jax.experimental.pallas import tpu as pltpu
from jax.experimental.pallas import tpu_sc as plsc

_N, _E, _D, _R = 10000, 160000, 256, 20
_H = _D // 2
_NS = 16
_CH = 40
_CPT = _E // (_NS * _CH)
_NG = 10
_CPG = _CPT // _NG
_NP = 10240
_RPT = _NP // _NS


def _pack_bf16_pair(lo_f32, hi_f32):
    lo_u = lax.bitcast_convert_type(lo_f32, jnp.uint32)
    hi_u = lax.bitcast_convert_type(hi_f32, jnp.uint32)
    lo_b = (lo_u + jnp.uint32(0x8000)) >> jnp.uint32(16)
    hi_b = (hi_u + jnp.uint32(0x8000)) & jnp.uint32(0xFFFF0000)
    return lax.bitcast_convert_type(hi_b | lo_b, jnp.int32)


def _silu(x):
    return 0.5 * x * (1.0 + jnp.tanh(0.5 * x))


def _edge_mlp_body(ea, w1, b1, w2, b2, out):
    x = jnp.dot(ea[...], w1[...], preferred_element_type=jnp.float32) + b1[...]
    x = _silu(x)
    y = jnp.dot(x.astype(jnp.bfloat16), w2[...],
                preferred_element_type=jnp.float32) + b2[...]
    y = _silu(y)
    out[0] = _pack_bf16_pair(y[:, 0:64], y[:, 64:128])
    out[1] = _pack_bf16_pair(y[:, 128:192], y[:, 192:256])


def _edge_mlp(edge_attr, W1, b1, W2, b2):
    Eb = 2000
    return pl.pallas_call(
        _edge_mlp_body,
        grid=(_E // Eb,),
        in_specs=[
            pl.BlockSpec((Eb, _R), lambda i: (i, 0)),
            pl.BlockSpec((_R, _D), lambda i: (0, 0)),
            pl.BlockSpec((1, _D), lambda i: (0, 0)),
            pl.BlockSpec((_D, _D), lambda i: (0, 0)),
            pl.BlockSpec((1, _D), lambda i: (0, 0)),
        ],
        out_specs=pl.BlockSpec((2, Eb, _H // 2), lambda i: (0, i, 0)),
        out_shape=jax.ShapeDtypeStruct((2, _E, _H // 2), jnp.int32),
    )(edge_attr.astype(jnp.bfloat16), W1.astype(jnp.bfloat16),
      b1.reshape(1, _D), W2.astype(jnp.bfloat16), b2.reshape(1, _D))


def _sc_body(hpk, ew, ridx_h, cidx_h, zrows, out, ridx, cidx, hbuf, ebuf, mbuf,
             acc, hsem, esem):
    c = lax.axis_index("c")
    s = lax.axis_index("s")
    pltpu.sync_copy(zrows, acc.at[pl.ds(s * _RPT, _RPT)])
    plsc.subcore_barrier()

    hi_mask = jnp.full((16,), -65536, jnp.int32)
    sh16 = jnp.full((16,), 16, jnp.int32)

    def _unpack_lo(v):
        return lax.bitcast_convert_type(lax.shift_left(v, sh16), jnp.float32)

    def _unpack_hi(v):
        return lax.bitcast_convert_type(lax.bitwise_and(v, hi_mask),
                                        jnp.float32)

    def _e0(g, j):
        return c * _E + (s * _CPT + g * _CPG + j) * _CH

    def _start(g, j, slot):
        pltpu.async_copy(ew.at[pl.ds(_e0(g, j), _CH)], ebuf.at[slot], esem)
        pltpu.async_copy(hpk.at[ridx.at[j]], hbuf.at[slot], hsem)

    def _finish(g, j, slot):
        pltpu.make_async_copy(ew.at[pl.ds(_e0(g, j), _CH)], ebuf.at[slot],
                              esem).wait()
        pltpu.make_async_copy(hpk.at[ridx.at[j]], hbuf.at[slot], hsem).wait()

    def _consume(g, j, slot):
        _finish(g, j, slot)

        def _mul_rows(h_unpack):
            def rowfn(r, carry2):
                for k in range(_H // 32):
                    lo_sl = pl.ds(k * 16, 16)
                    hi_sl = pl.ds(64 + k * 16, 16)
                    ep = ebuf[slot, r, lo_sl]
                    ha = h_unpack(hbuf[slot, r, lo_sl])
                    hb = h_unpack(hbuf[slot, r, hi_sl])
                    mbuf[r, lo_sl] = ha * _unpack_lo(ep)
                    mbuf[r, hi_sl] = hb * _unpack_hi(ep)
                return carry2

            lax.fori_loop(0, _CH, rowfn, 0)

        @pl.when(c == 0)
        def _():
            _mul_rows(_unpack_lo)

        @pl.when(c == 1)
        def _():
            _mul_rows(_unpack_hi)

        pltpu.sync_copy(mbuf, acc.at[cidx.at[j]], add=True)

    def group(g, carry):
        pltpu.sync_copy(ridx_h.at[s, g], ridx)
        pltpu.sync_copy(cidx_h.at[s, g], cidx)
        _start(g, 0, 0)
        _start(g, 1, 1)

        def pair(p, carry1):
            j0 = 2 * p
            _consume(g, j0, 0)

            @pl.when(j0 + 2 < _CPG)
            def _():
                _start(g, j0 + 2, 0)

            _consume(g, j0 + 1, 1)

            @pl.when(j0 + 3 < _CPG)
            def _():
                _start(g, j0 + 3, 1)

            return carry1

        lax.fori_loop(0, _CPG // 2, pair, 0)
        if _CPG % 2:
            _consume(g, _CPG - 1, 0)
        return carry

    lax.fori_loop(0, _NG, group, 0)
    plsc.subcore_barrier()
    pltpu.sync_copy(acc.at[pl.ds(s * _RPT, _RPT)], out.at[c, s])


def _sc_scatter(h2, ew, ridx_h, cidx_h, zrows):
    mesh = plsc.VectorSubcoreMesh(core_axis_name="c", subcore_axis_name="s")
    run = functools.partial(
        pl.kernel,
        mesh=mesh,
        out_type=jax.ShapeDtypeStruct((2, _NS, _RPT, _H), jnp.float32),
        scratch_types=[
            pltpu.VMEM((_CPG, _CH), jnp.int32),
            pltpu.VMEM((_CPG, _CH), jnp.int32),
            pltpu.VMEM((2, _CH, _H), jnp.int32),
            pltpu.VMEM((2, _CH, _H // 2), jnp.int32),
            pltpu.VMEM((_CH, _H), jnp.float32),
            pltpu.VMEM_SHARED((_NP, _H), jnp.float32),
            pltpu.SemaphoreType.DMA,
            pltpu.SemaphoreType.DMA,
        ],
    )(_sc_body)
    return run(h2, ew, ridx_h, cidx_h, zrows)


def _out_mlp_body(h, n0, n1, w3h, w3n0, w3n1, b3, w4, b4, o):
    t = (jnp.dot(h[...], w3h[...], preferred_element_type=jnp.float32)
         + jnp.dot(n0[...], w3n0[...], preferred_element_type=jnp.float32)
         + jnp.dot(n1[...], w3n1[...], preferred_element_type=jnp.float32)
         + b3[...])
    t = _silu(t)
    o[...] = h[...] + jnp.dot(t, w4[...], preferred_element_type=jnp.float32) + b4[...]


def _out_mlp(h, n0, n1, W3, b3, W4, b4):
    Nb = 2000
    full = lambda i: (0, 0)
    return pl.pallas_call(
        _out_mlp_body,
        grid=(_N // Nb,),
        in_specs=[
            pl.BlockSpec((Nb, _D), lambda i: (i, 0)),
            pl.BlockSpec((Nb, _H), lambda i: (i, 0)),
            pl.BlockSpec((Nb, _H), lambda i: (i, 0)),
            pl.BlockSpec((_D, _D), full),
            pl.BlockSpec((_H, _D), full),
            pl.BlockSpec((_H, _D), full),
            pl.BlockSpec((1, _D), full),
            pl.BlockSpec((_D, _D), full),
            pl.BlockSpec((1, _D), full),
        ],
        out_specs=pl.BlockSpec((Nb, _D), lambda i: (i, 0)),
        out_shape=jax.ShapeDtypeStruct((_N, _D), jnp.float32),
    )(h, n0, n1, W3[:_D], W3[_D:_D + _H], W3[_D + _H:],
      b3.reshape(1, _D), W4, b4.reshape(1, _D))


def kernel(h, edge_index, edge_attr, mask, W1, b1, W2, b2, W3, b3, W4, b4):
    row = edge_index[0]
    col = edge_index[1]
    ew2 = _edge_mlp(edge_attr, W1, b1, W2, b2).reshape(2 * _E, _H // 2)
    hpk = _pack_bf16_pair(h[:, :_H], h[:, _H:])
    ridx_h = row.reshape(_NS, _NG, _CPG, _CH)
    cidx_h = col.reshape(_NS, _NG, _CPG, _CH)
    zrows = jnp.zeros((_RPT, _H), jnp.float32)
    hn4 = _sc_scatter(hpk, ew2, ridx_h, cidx_h, zrows)
    hn = hn4.reshape(2, _NP, _H)[:, :_N]
    return _out_mlp(h, hn[0], hn[1], W3, b3, W4, b4)


# speedup vs baseline: 1.0600x; 1.0600x over previous
import functools

import jax
import jax.numpy as jnp
import numpy as np
from jax import lax
from jax.experimental import pallas as pl
from jax.experimental.pallas import tpu as pltpu
from jax.experimental.pallas import tpu_sc as plsc

_N, _E, _D, _R = 10000, 160000, 256, 20
_H = _D // 2
_NS = 16
_CH = 40
_CPT = _E // (_NS * _CH)
_NG = 10
_CPG = _CPT // _NG
_NP = 10240
_RPT = _NP // _NS


def _pack_bf16_pair(lo_f32, hi_f32):
    lo_u = lax.bitcast_convert_type(lo_f32, jnp.uint32)
    hi_u = lax.bitcast_convert_type(hi_f32, jnp.uint32)
    lo_b = (lo_u + jnp.uint32(0x8000)) >> jnp.uint32(16)
    hi_b = (hi_u + jnp.uint32(0x8000)) & jnp.uint32(0xFFFF0000)
    return lax.bitcast_convert_type(hi_b | lo_b, jnp.int32)


def _silu(x):
    return 0.5 * x * (1.0 + jnp.tanh(0.5 * x))


def _edge_mlp_body(ea, w1, b1, w2, b2, out):
    x = jnp.dot(ea[...].astype(jnp.bfloat16), w1[...],
                preferred_element_type=jnp.float32) + b1[...]
    x = _silu(x)
    y = jnp.dot(x.astype(jnp.bfloat16), w2[...],
                preferred_element_type=jnp.float32) + b2[...]
    y = _silu(y)
    out[0] = _pack_bf16_pair(y[:, 0:64], y[:, 64:128])
    out[1] = _pack_bf16_pair(y[:, 128:192], y[:, 192:256])


def _edge_mlp(edge_attr, W1, b1, W2, b2):
    Eb = 4000
    return pl.pallas_call(
        _edge_mlp_body,
        grid=(_E // Eb,),
        in_specs=[
            pl.BlockSpec((Eb, _R), lambda i: (i, 0)),
            pl.BlockSpec((_R, _D), lambda i: (0, 0)),
            pl.BlockSpec((1, _D), lambda i: (0, 0)),
            pl.BlockSpec((_D, _D), lambda i: (0, 0)),
            pl.BlockSpec((1, _D), lambda i: (0, 0)),
        ],
        out_specs=pl.BlockSpec((2, Eb, _H // 2), lambda i: (0, i, 0)),
        out_shape=jax.ShapeDtypeStruct((2, _E, _H // 2), jnp.int32),
    )(edge_attr, W1.astype(jnp.bfloat16),
      b1.reshape(1, _D), W2.astype(jnp.bfloat16), b2.reshape(1, _D))


def _sc_body(hpk, ew, ridx_h, cidx_h, zrows, out, ridx, cidx, hbuf, ebuf, mbuf,
             acc, hsem, esem, ssem):
    c = lax.axis_index("c")
    s = lax.axis_index("s")
    pltpu.sync_copy(zrows, acc.at[pl.ds(s * _RPT, _RPT)])
    plsc.subcore_barrier()
    pltpu.async_copy(mbuf.at[0], out.at[c, s, pl.ds(0, _CH)], ssem)
    pltpu.async_copy(mbuf.at[1], out.at[c, s, pl.ds(0, _CH)], ssem)

    hi_mask = jnp.full((16,), -65536, jnp.int32)
    sh16 = jnp.full((16,), 16, jnp.int32)

    def _unpack_lo(v):
        return lax.bitcast_convert_type(lax.shift_left(v, sh16), jnp.float32)

    def _unpack_hi(v):
        return lax.bitcast_convert_type(lax.bitwise_and(v, hi_mask),
                                        jnp.float32)

    def _e0(g, j):
        return c * _E + (s * _CPT + g * _CPG + j) * _CH

    def _start(g, j, slot):
        pltpu.async_copy(ew.at[pl.ds(_e0(g, j), _CH)], ebuf.at[slot], esem)
        pltpu.async_copy(hpk.at[ridx.at[j]], hbuf.at[slot], hsem)

    def _finish(g, j, slot):
        pltpu.make_async_copy(ew.at[pl.ds(_e0(g, j), _CH)], ebuf.at[slot],
                              esem).wait()
        pltpu.make_async_copy(hpk.at[ridx.at[j]], hbuf.at[slot], hsem).wait()

    def _consume(g, j, slot):
        _finish(g, j, slot)
        pltpu.make_async_copy(mbuf.at[slot], acc.at[cidx.at[j]], ssem).wait()

        def _mul_rows(h_unpack):
            def rowfn(r, carry2):
                for k in range(_H // 32):
                    lo_sl = pl.ds(k * 16, 16)
                    hi_sl = pl.ds(64 + k * 16, 16)
                    ep = ebuf[slot, r, lo_sl]
                    ha = h_unpack(hbuf[slot, r, lo_sl])
                    hb = h_unpack(hbuf[slot, r, hi_sl])
                    mbuf[slot, r, lo_sl] = ha * _unpack_lo(ep)
                    mbuf[slot, r, hi_sl] = hb * _unpack_hi(ep)
                return carry2

            lax.fori_loop(0, _CH, rowfn, 0)

        @pl.when(c == 0)
        def _():
            _mul_rows(_unpack_lo)

        @pl.when(c == 1)
        def _():
            _mul_rows(_unpack_hi)

        pltpu.async_copy(mbuf.at[slot], acc.at[cidx.at[j]], ssem, add=True)

    def group(g, carry):
        pltpu.sync_copy(ridx_h.at[s, g], ridx)
        pltpu.sync_copy(cidx_h.at[s, g], cidx)
        _start(g, 0, 0)
        _start(g, 1, 1)

        def pair(p, carry1):
            j0 = 2 * p
            _consume(g, j0, 0)

            @pl.when(j0 + 2 < _CPG)
            def _():
                _start(g, j0 + 2, 0)

            _consume(g, j0 + 1, 1)

            @pl.when(j0 + 3 < _CPG)
            def _():
                _start(g, j0 + 3, 1)

            return carry1

        lax.fori_loop(0, _CPG // 2, pair, 0)
        if _CPG % 2:
            _consume(g, _CPG - 1, 0)
        return carry

    lax.fori_loop(0, _NG, group, 0)
    pltpu.make_async_copy(mbuf.at[0], acc.at[cidx.at[0]], ssem).wait()
    pltpu.make_async_copy(mbuf.at[1], acc.at[cidx.at[0]], ssem).wait()
    plsc.subcore_barrier()
    pltpu.sync_copy(acc.at[pl.ds(s * _RPT, _RPT)], out.at[c, s])


def _sc_scatter(h2, ew, ridx_h, cidx_h, zrows):
    mesh = plsc.VectorSubcoreMesh(core_axis_name="c", subcore_axis_name="s")
    run = functools.partial(
        pl.kernel,
        mesh=mesh,
        out_type=jax.ShapeDtypeStruct((2, _NS, _RPT, _H), jnp.float32),
        scratch_types=[
            pltpu.VMEM((_CPG, _CH), jnp.int32),
            pltpu.VMEM((_CPG, _CH), jnp.int32),
            pltpu.VMEM((2, _CH, _H), jnp.int32),
            pltpu.VMEM((2, _CH, _H // 2), jnp.int32),
            pltpu.VMEM((2, _CH, _H), jnp.float32),
            pltpu.VMEM_SHARED((_NP, _H), jnp.float32),
            pltpu.SemaphoreType.DMA,
            pltpu.SemaphoreType.DMA,
            pltpu.SemaphoreType.DMA,
        ],
    )(_sc_body)
    return run(h2, ew, ridx_h, cidx_h, zrows)


def _out_mlp_body(h, n0, n1, w3h, w3n0, w3n1, b3, w4, b4, o):
    t = (jnp.dot(h[...], w3h[...], preferred_element_type=jnp.float32)
         + jnp.dot(n0[...], w3n0[...], preferred_element_type=jnp.float32)
         + jnp.dot(n1[...], w3n1[...], preferred_element_type=jnp.float32)
         + b3[...])
    t = _silu(t)
    o[...] = h[...] + jnp.dot(t, w4[...], preferred_element_type=jnp.float32) + b4[...]


def _out_mlp(h, n0, n1, W3, b3, W4, b4):
    Nb = 2000
    full = lambda i: (0, 0)
    return pl.pallas_call(
        _out_mlp_body,
        grid=(_N // Nb,),
        in_specs=[
            pl.BlockSpec((Nb, _D), lambda i: (i, 0)),
            pl.BlockSpec((Nb, _H), lambda i: (i, 0)),
            pl.BlockSpec((Nb, _H), lambda i: (i, 0)),
            pl.BlockSpec((_D, _D), full),
            pl.BlockSpec((_H, _D), full),
            pl.BlockSpec((_H, _D), full),
            pl.BlockSpec((1, _D), full),
            pl.BlockSpec((_D, _D), full),
            pl.BlockSpec((1, _D), full),
        ],
        out_specs=pl.BlockSpec((Nb, _D), lambda i: (i, 0)),
        out_shape=jax.ShapeDtypeStruct((_N, _D), jnp.float32),
    )(h, n0, n1, W3[:_D], W3[_D:_D + _H], W3[_D + _H:],
      b3.reshape(1, _D), W4, b4.reshape(1, _D))


def kernel(h, edge_index, edge_attr, mask, W1, b1, W2, b2, W3, b3, W4, b4):
    row = edge_index[0]
    col = edge_index[1]
    ew2 = _edge_mlp(edge_attr, W1, b1, W2, b2).reshape(2 * _E, _H // 2)
    hpk = _pack_bf16_pair(h[:, :_H], h[:, _H:])
    ridx_h = row.reshape(_NS, _NG, _CPG, _CH)
    cidx_h = col.reshape(_NS, _NG, _CPG, _CH)
    zrows = jnp.zeros((_RPT, _H), jnp.float32)
    hn4 = _sc_scatter(hpk, ew2, ridx_h, cidx_h, zrows)
    hn = hn4.reshape(2, _NP, _H)[:, :_N]
    return _out_mlp(h, hn[0], hn[1], W3, b3, W4, b4)


# speedup vs baseline: 1.1672x; 1.1012x over previous
import functools

import jax
import jax.numpy as jnp
import numpy as np
from jax import lax
from jax.experimental import pallas as pl
from jax.experimental.pallas import tpu as pltpu
from jax.experimental.pallas import tpu_sc as plsc

_N, _E, _D, _R = 10000, 160000, 256, 20
_H = _D // 2
_NS = 16
_CH = 40
_CPT = _E // (_NS * _CH)
_NG = 10
_CPG = _CPT // _NG
_NP = 10240
_RPT = _NP // _NS


def _pack_bf16_pair(lo_f32, hi_f32):
    lo_u = lax.bitcast_convert_type(lo_f32, jnp.uint32)
    hi_u = lax.bitcast_convert_type(hi_f32, jnp.uint32)
    lo_b = (lo_u + jnp.uint32(0x8000)) >> jnp.uint32(16)
    hi_b = (hi_u + jnp.uint32(0x8000)) & jnp.uint32(0xFFFF0000)
    return lax.bitcast_convert_type(hi_b | lo_b, jnp.int32)


def _silu(x):
    return 0.5 * x * (1.0 + jnp.tanh(0.5 * x))


def _edge_mlp_body(eat, w1, b1, w2, b2, out):
    x = lax.dot_general(eat[...].astype(jnp.bfloat16), w1[...],
                        (((0,), (0,)), ((), ())),
                        preferred_element_type=jnp.float32) + b1[...]
    x = _silu(x)
    y = jnp.dot(x.astype(jnp.bfloat16), w2[...],
                preferred_element_type=jnp.float32) + b2[...]
    y = _silu(y)
    out[0] = _pack_bf16_pair(y[:, 0:64], y[:, 64:128])
    out[1] = _pack_bf16_pair(y[:, 128:192], y[:, 192:256])


def _edge_mlp(edge_attr, W1, b1, W2, b2):
    Eb = 6400
    return pl.pallas_call(
        _edge_mlp_body,
        grid=(_E // Eb,),
        in_specs=[
            pl.BlockSpec((_R, Eb), lambda i: (0, i)),
            pl.BlockSpec((_R, _D), lambda i: (0, 0)),
            pl.BlockSpec((1, _D), lambda i: (0, 0)),
            pl.BlockSpec((_D, _D), lambda i: (0, 0)),
            pl.BlockSpec((1, _D), lambda i: (0, 0)),
        ],
        out_specs=pl.BlockSpec((2, Eb, _H // 2), lambda i: (0, i, 0)),
        out_shape=jax.ShapeDtypeStruct((2, _E, _H // 2), jnp.int32),
    )(edge_attr.T, W1.astype(jnp.bfloat16),
      b1.reshape(1, _D), W2.astype(jnp.bfloat16), b2.reshape(1, _D))


def _sc_body(hpk, ew, ridx_h, cidx_h, zrows, out, ridx, cidx, hbuf, ebuf, mbuf,
             acc, hsem, esem, ssem):
    c = lax.axis_index("c")
    s = lax.axis_index("s")
    pltpu.sync_copy(zrows, acc.at[pl.ds(s * _RPT, _RPT)])
    plsc.subcore_barrier()
    pltpu.async_copy(mbuf.at[0], out.at[c, s, pl.ds(0, _CH)], ssem)
    pltpu.async_copy(mbuf.at[1], out.at[c, s, pl.ds(0, _CH)], ssem)

    hi_mask = jnp.full((16,), -65536, jnp.int32)
    sh16 = jnp.full((16,), 16, jnp.int32)

    def _unpack_lo(v):
        return lax.bitcast_convert_type(lax.shift_left(v, sh16), jnp.float32)

    def _unpack_hi(v):
        return lax.bitcast_convert_type(lax.bitwise_and(v, hi_mask),
                                        jnp.float32)

    def _e0(g, j):
        return c * _E + (s * _CPT + g * _CPG + j) * _CH

    def _start(g, j, slot):
        pltpu.async_copy(ew.at[pl.ds(_e0(g, j), _CH)], ebuf.at[slot], esem)
        pltpu.async_copy(hpk.at[ridx.at[j]], hbuf.at[slot], hsem)

    def _finish(g, j, slot):
        pltpu.make_async_copy(ew.at[pl.ds(_e0(g, j), _CH)], ebuf.at[slot],
                              esem).wait()
        pltpu.make_async_copy(hpk.at[ridx.at[j]], hbuf.at[slot], hsem).wait()

    def _consume(g, j, slot):
        _finish(g, j, slot)
        pltpu.make_async_copy(mbuf.at[slot], acc.at[cidx.at[j]], ssem).wait()

        def _mul_rows(h_unpack):
            def rowfn(q, carry2):
                for rr in range(4):
                    r = 4 * q + rr
                    for k in range(_H // 32):
                        lo_sl = pl.ds(k * 16, 16)
                        hi_sl = pl.ds(64 + k * 16, 16)
                        ep = ebuf[slot, r, lo_sl]
                        ha = h_unpack(hbuf[slot, r, lo_sl])
                        hb = h_unpack(hbuf[slot, r, hi_sl])
                        mbuf[slot, r, lo_sl] = ha * _unpack_lo(ep)
                        mbuf[slot, r, hi_sl] = hb * _unpack_hi(ep)
                return carry2

            lax.fori_loop(0, _CH // 4, rowfn, 0)

        @pl.when(c == 0)
        def _():
            _mul_rows(_unpack_lo)

        @pl.when(c == 1)
        def _():
            _mul_rows(_unpack_hi)

        pltpu.async_copy(mbuf.at[slot], acc.at[cidx.at[j]], ssem, add=True)

    def group(g, carry):
        pltpu.sync_copy(ridx_h.at[s, g], ridx)
        pltpu.sync_copy(cidx_h.at[s, g], cidx)
        _start(g, 0, 0)
        _start(g, 1, 1)

        def pair(p, carry1):
            j0 = 2 * p
            _consume(g, j0, 0)

            @pl.when(j0 + 2 < _CPG)
            def _():
                _start(g, j0 + 2, 0)

            _consume(g, j0 + 1, 1)

            @pl.when(j0 + 3 < _CPG)
            def _():
                _start(g, j0 + 3, 1)

            return carry1

        lax.fori_loop(0, _CPG // 2, pair, 0)
        if _CPG % 2:
            _consume(g, _CPG - 1, 0)
        return carry

    lax.fori_loop(0, _NG, group, 0)
    pltpu.make_async_copy(mbuf.at[0], acc.at[cidx.at[0]], ssem).wait()
    pltpu.make_async_copy(mbuf.at[1], acc.at[cidx.at[0]], ssem).wait()
    plsc.subcore_barrier()
    pltpu.sync_copy(acc.at[pl.ds(s * _RPT, _RPT)], out.at[c, s])


def _sc_scatter(h2, ew, ridx_h, cidx_h, zrows):
    mesh = plsc.VectorSubcoreMesh(core_axis_name="c", subcore_axis_name="s")
    run = functools.partial(
        pl.kernel,
        mesh=mesh,
        out_type=jax.ShapeDtypeStruct((2, _NS, _RPT, _H), jnp.float32),
        scratch_types=[
            pltpu.VMEM((_CPG, _CH), jnp.int32),
            pltpu.VMEM((_CPG, _CH), jnp.int32),
            pltpu.VMEM((2, _CH, _H), jnp.int32),
            pltpu.VMEM((2, _CH, _H // 2), jnp.int32),
            pltpu.VMEM((2, _CH, _H), jnp.float32),
            pltpu.VMEM_SHARED((_NP, _H), jnp.float32),
            pltpu.SemaphoreType.DMA,
            pltpu.SemaphoreType.DMA,
            pltpu.SemaphoreType.DMA,
        ],
    )(_sc_body)
    return run(h2, ew, ridx_h, cidx_h, zrows)


def _out_mlp_body(h, n0, n1, w3h, w3n0, w3n1, b3, w4, b4, o):
    t = (jnp.dot(h[...], w3h[...], preferred_element_type=jnp.float32)
         + jnp.dot(n0[...], w3n0[...], preferred_element_type=jnp.float32)
         + jnp.dot(n1[...], w3n1[...], preferred_element_type=jnp.float32)
         + b3[...])
    t = _silu(t)
    o[...] = h[...] + jnp.dot(t, w4[...], preferred_element_type=jnp.float32) + b4[...]


def _out_mlp(h, n0, n1, W3, b3, W4, b4):
    Nb = 2000
    full = lambda i: (0, 0)
    return pl.pallas_call(
        _out_mlp_body,
        grid=(_N // Nb,),
        in_specs=[
            pl.BlockSpec((Nb, _D), lambda i: (i, 0)),
            pl.BlockSpec((Nb, _H), lambda i: (i, 0)),
            pl.BlockSpec((Nb, _H), lambda i: (i, 0)),
            pl.BlockSpec((_D, _D), full),
            pl.BlockSpec((_H, _D), full),
            pl.BlockSpec((_H, _D), full),
            pl.BlockSpec((1, _D), full),
            pl.BlockSpec((_D, _D), full),
            pl.BlockSpec((1, _D), full),
        ],
        out_specs=pl.BlockSpec((Nb, _D), lambda i: (i, 0)),
        out_shape=jax.ShapeDtypeStruct((_N, _D), jnp.float32),
    )(h, n0, n1, W3[:_D], W3[_D:_D + _H], W3[_D + _H:],
      b3.reshape(1, _D), W4, b4.reshape(1, _D))


def kernel(h, edge_index, edge_attr, mask, W1, b1, W2, b2, W3, b3, W4, b4):
    row = edge_index[0]
    col = edge_index[1]
    ew2 = _edge_mlp(edge_attr, W1, b1, W2, b2).reshape(2 * _E, _H // 2)
    hpk = _pack_bf16_pair(h[:, :_H], h[:, _H:])
    ridx_h = row.reshape(_NS, _NG, _CPG, _CH)
    cidx_h = col.reshape(_NS, _NG, _CPG, _CH)
    zrows = jnp.zeros((_RPT, _H), jnp.float32)
    hn4 = _sc_scatter(hpk, ew2, ridx_h, cidx_h, zrows)
    hn = hn4.reshape(2, _NP, _H)[:, :_N]
    return _out_mlp(h, hn[0], hn[1], W3, b3, W4, b4)


# speedup vs baseline: 1.2540x; 1.0744x over previous
import functools

import jax
import jax.numpy as jnp
import numpy as np
from jax import lax
from jax.experimental import pallas as pl
from jax.experimental.pallas import tpu as pltpu
from jax.experimental.pallas import tpu_sc as plsc

_N, _E, _D, _R = 10000, 160000, 256, 20
_H = _D // 2
_NS = 16
_NSL = 2
_ES = _E // _NSL
_CH = 40
_CPT = _ES // (_NS * _CH)
_NG = 5
_CPG = _CPT // _NG
_NP = 10240
_RPT = _NP // _NS


def _pack_bf16_pair(lo_f32, hi_f32):
    lo_u = lax.bitcast_convert_type(lo_f32, jnp.uint32)
    hi_u = lax.bitcast_convert_type(hi_f32, jnp.uint32)
    lo_b = (lo_u + jnp.uint32(0x8000)) >> jnp.uint32(16)
    hi_b = (hi_u + jnp.uint32(0x8000)) & jnp.uint32(0xFFFF0000)
    return lax.bitcast_convert_type(hi_b | lo_b, jnp.int32)


def _silu(x):
    return 0.5 * x * (1.0 + jnp.tanh(0.5 * x))


def _edge_mlp_body(eat, w1, b1, w2, b2, out):
    x = lax.dot_general(eat[...].astype(jnp.bfloat16), w1[...],
                        (((0,), (0,)), ((), ())),
                        preferred_element_type=jnp.float32) + b1[...]
    x = _silu(x)
    y = jnp.dot(x.astype(jnp.bfloat16), w2[...],
                preferred_element_type=jnp.float32) + b2[...]
    y = _silu(y)
    out[0] = _pack_bf16_pair(y[:, 0:64], y[:, 64:128])
    out[1] = _pack_bf16_pair(y[:, 128:192], y[:, 192:256])


def _edge_mlp(eat, W1b, b1, W2b, b2, sl):
    Eb = 3200
    nb = _ES // Eb
    return pl.pallas_call(
        _edge_mlp_body,
        grid=(nb,),
        in_specs=[
            pl.BlockSpec((_R, Eb), lambda i: (0, sl * nb + i)),
            pl.BlockSpec((_R, _D), lambda i: (0, 0)),
            pl.BlockSpec((1, _D), lambda i: (0, 0)),
            pl.BlockSpec((_D, _D), lambda i: (0, 0)),
            pl.BlockSpec((1, _D), lambda i: (0, 0)),
        ],
        out_specs=pl.BlockSpec((2, Eb, _H // 2), lambda i: (0, i, 0)),
        out_shape=jax.ShapeDtypeStruct((2, _ES, _H // 2), jnp.int32),
    )(eat, W1b, b1, W2b, b2)


def _sc_body(hpk, ew, ridx_h, cidx_h, init, out, ridx, cidx, hbuf, ebuf, mbuf,
             acc, hsem, esem, ssem, chained):
    c = lax.axis_index("c")
    s = lax.axis_index("s")
    if chained:
        pltpu.sync_copy(init.at[c, s], acc.at[pl.ds(s * _RPT, _RPT)])
    else:
        pltpu.sync_copy(init, acc.at[pl.ds(s * _RPT, _RPT)])
    plsc.subcore_barrier()
    pltpu.async_copy(mbuf.at[0], out.at[c, s, pl.ds(0, _CH)], ssem)
    pltpu.async_copy(mbuf.at[1], out.at[c, s, pl.ds(0, _CH)], ssem)

    hi_mask = jnp.full((16,), -65536, jnp.int32)
    sh16 = jnp.full((16,), 16, jnp.int32)

    def _unpack_lo(v):
        return lax.bitcast_convert_type(lax.shift_left(v, sh16), jnp.float32)

    def _unpack_hi(v):
        return lax.bitcast_convert_type(lax.bitwise_and(v, hi_mask),
                                        jnp.float32)

    def _e0(g, j):
        return c * _ES + (s * _CPT + g * _CPG + j) * _CH

    def _start(g, j, slot):
        pltpu.async_copy(ew.at[pl.ds(_e0(g, j), _CH)], ebuf.at[slot], esem)
        pltpu.async_copy(hpk.at[ridx.at[j]], hbuf.at[slot], hsem)

    def _finish(g, j, slot):
        pltpu.make_async_copy(ew.at[pl.ds(_e0(g, j), _CH)], ebuf.at[slot],
                              esem).wait()
        pltpu.make_async_copy(hpk.at[ridx.at[j]], hbuf.at[slot], hsem).wait()

    def _consume(g, j, slot):
        _finish(g, j, slot)
        pltpu.make_async_copy(mbuf.at[slot], acc.at[cidx.at[j]], ssem).wait()

        def _mul_rows(h_unpack):
            def rowfn(q, carry2):
                for rr in range(4):
                    r = 4 * q + rr
                    for k in range(_H // 32):
                        lo_sl = pl.ds(k * 16, 16)
                        hi_sl = pl.ds(64 + k * 16, 16)
                        ep = ebuf[slot, r, lo_sl]
                        ha = h_unpack(hbuf[slot, r, lo_sl])
                        hb = h_unpack(hbuf[slot, r, hi_sl])
                        mbuf[slot, r, lo_sl] = ha * _unpack_lo(ep)
                        mbuf[slot, r, hi_sl] = hb * _unpack_hi(ep)
                return carry2

            lax.fori_loop(0, _CH // 4, rowfn, 0)

        @pl.when(c == 0)
        def _():
            _mul_rows(_unpack_lo)

        @pl.when(c == 1)
        def _():
            _mul_rows(_unpack_hi)

        pltpu.async_copy(mbuf.at[slot], acc.at[cidx.at[j]], ssem, add=True)

    def group(g, carry):
        pltpu.sync_copy(ridx_h.at[s, g], ridx)
        pltpu.sync_copy(cidx_h.at[s, g], cidx)
        _start(g, 0, 0)
        _start(g, 1, 1)

        def pair(p, carry1):
            j0 = 2 * p
            _consume(g, j0, 0)

            @pl.when(j0 + 2 < _CPG)
            def _():
                _start(g, j0 + 2, 0)

            _consume(g, j0 + 1, 1)

            @pl.when(j0 + 3 < _CPG)
            def _():
                _start(g, j0 + 3, 1)

            return carry1

        lax.fori_loop(0, _CPG // 2, pair, 0)
        if _CPG % 2:
            _consume(g, _CPG - 1, 0)
        return carry

    lax.fori_loop(0, _NG, group, 0)
    pltpu.make_async_copy(mbuf.at[0], acc.at[cidx.at[0]], ssem).wait()
    pltpu.make_async_copy(mbuf.at[1], acc.at[cidx.at[0]], ssem).wait()
    plsc.subcore_barrier()
    pltpu.sync_copy(acc.at[pl.ds(s * _RPT, _RPT)], out.at[c, s])


def _sc_scatter(h2, ew, ridx_h, cidx_h, init, chained):
    mesh = plsc.VectorSubcoreMesh(core_axis_name="c", subcore_axis_name="s")
    run = functools.partial(
        pl.kernel,
        mesh=mesh,
        out_type=jax.ShapeDtypeStruct((2, _NS, _RPT, _H), jnp.float32),
        scratch_types=[
            pltpu.VMEM((_CPG, _CH), jnp.int32),
            pltpu.VMEM((_CPG, _CH), jnp.int32),
            pltpu.VMEM((2, _CH, _H), jnp.int32),
            pltpu.VMEM((2, _CH, _H // 2), jnp.int32),
            pltpu.VMEM((2, _CH, _H), jnp.float32),
            pltpu.VMEM_SHARED((_NP, _H), jnp.float32),
            pltpu.SemaphoreType.DMA,
            pltpu.SemaphoreType.DMA,
            pltpu.SemaphoreType.DMA,
        ],
    )(functools.partial(_sc_body, chained=chained))
    return run(h2, ew, ridx_h, cidx_h, init)


def _out_mlp_body(h, n0, n1, w3h, w3n0, w3n1, b3, w4, b4, o):
    t = (jnp.dot(h[...], w3h[...], preferred_element_type=jnp.float32)
         + jnp.dot(n0[...], w3n0[...], preferred_element_type=jnp.float32)
         + jnp.dot(n1[...], w3n1[...], preferred_element_type=jnp.float32)
         + b3[...])
    t = _silu(t)
    o[...] = h[...] + jnp.dot(t, w4[...], preferred_element_type=jnp.float32) + b4[...]


def _out_mlp(h, n0, n1, W3, b3, W4, b4):
    Nb = 2000
    full = lambda i: (0, 0)
    return pl.pallas_call(
        _out_mlp_body,
        grid=(_N // Nb,),
        in_specs=[
            pl.BlockSpec((Nb, _D), lambda i: (i, 0)),
            pl.BlockSpec((Nb, _H), lambda i: (i, 0)),
            pl.BlockSpec((Nb, _H), lambda i: (i, 0)),
            pl.BlockSpec((_D, _D), full),
            pl.BlockSpec((_H, _D), full),
            pl.BlockSpec((_H, _D), full),
            pl.BlockSpec((1, _D), full),
            pl.BlockSpec((_D, _D), full),
            pl.BlockSpec((1, _D), full),
        ],
        out_specs=pl.BlockSpec((Nb, _D), lambda i: (i, 0)),
        out_shape=jax.ShapeDtypeStruct((_N, _D), jnp.float32),
    )(h, n0, n1, W3[:_D], W3[_D:_D + _H], W3[_D + _H:],
      b3.reshape(1, _D), W4, b4.reshape(1, _D))


def kernel(h, edge_index, edge_attr, mask, W1, b1, W2, b2, W3, b3, W4, b4):
    row = edge_index[0]
    col = edge_index[1]
    hpk = _pack_bf16_pair(h[:, :_H], h[:, _H:])
    eat = edge_attr.T
    W1b = W1.astype(jnp.bfloat16)
    W2b = W2.astype(jnp.bfloat16)
    b1r = b1.reshape(1, _D)
    b2r = b2.reshape(1, _D)
    zrows = jnp.zeros((_RPT, _H), jnp.float32)
    hn4 = None
    for sl in range(_NSL):
        ew2 = _edge_mlp(eat, W1b, b1r, W2b, b2r, sl).reshape(2 * _ES, _H // 2)
        e0, e1 = sl * _ES, (sl + 1) * _ES
        ridx_h = row[e0:e1].reshape(_NS, _NG, _CPG, _CH)
        cidx_h = col[e0:e1].reshape(_NS, _NG, _CPG, _CH)
        init = zrows if hn4 is None else hn4
        hn4 = _sc_scatter(hpk, ew2, ridx_h, cidx_h, init, chained=sl > 0)
    hn = hn4.reshape(2, _NP, _H)[:, :_N]
    return _out_mlp(h, hn[0], hn[1], W3, b3, W4, b4)


# speedup vs baseline: 1.2802x; 1.0209x over previous
import functools

import jax
import jax.numpy as jnp
import numpy as np
from jax import lax
from jax.experimental import pallas as pl
from jax.experimental.pallas import tpu as pltpu
from jax.experimental.pallas import tpu_sc as plsc

_N, _E, _D, _R = 10000, 160000, 256, 20
_H = _D // 2
_NS = 16
_CH = 40
_SLICES = ((57600, 6), (102400, 8))
assert sum(e for e, _ in _SLICES) == _E
_NP = 10240
_RPT = _NP // _NS


def _pack_bf16_pair(lo_f32, hi_f32):
    lo_u = lax.bitcast_convert_type(lo_f32, jnp.uint32)
    hi_u = lax.bitcast_convert_type(hi_f32, jnp.uint32)
    lo_b = (lo_u + jnp.uint32(0x8000)) >> jnp.uint32(16)
    hi_b = (hi_u + jnp.uint32(0x8000)) & jnp.uint32(0xFFFF0000)
    return lax.bitcast_convert_type(hi_b | lo_b, jnp.int32)


def _silu(x):
    return 0.5 * x * (1.0 + jnp.tanh(0.5 * x))


def _edge_mlp_body(eat, w1, b1, w2, b2, out):
    x = lax.dot_general(eat[...].astype(jnp.bfloat16), w1[...],
                        (((0,), (0,)), ((), ())),
                        preferred_element_type=jnp.float32) + b1[...]
    x = _silu(x)
    y = jnp.dot(x.astype(jnp.bfloat16), w2[...],
                preferred_element_type=jnp.float32) + b2[...]
    y = _silu(y)
    out[0] = _pack_bf16_pair(y[:, 0:64], y[:, 64:128])
    out[1] = _pack_bf16_pair(y[:, 128:192], y[:, 192:256])


def _edge_mlp(eat, W1b, b1, W2b, b2, e0, es):
    Eb = 3200
    nb = es // Eb
    b0 = e0 // Eb
    return pl.pallas_call(
        _edge_mlp_body,
        grid=(nb,),
        in_specs=[
            pl.BlockSpec((_R, Eb), lambda i: (0, b0 + i)),
            pl.BlockSpec((_R, _D), lambda i: (0, 0)),
            pl.BlockSpec((1, _D), lambda i: (0, 0)),
            pl.BlockSpec((_D, _D), lambda i: (0, 0)),
            pl.BlockSpec((1, _D), lambda i: (0, 0)),
        ],
        out_specs=pl.BlockSpec((2, Eb, _H // 2), lambda i: (0, i, 0)),
        out_shape=jax.ShapeDtypeStruct((2, es, _H // 2), jnp.int32),
    )(eat, W1b, b1, W2b, b2)


def _sc_body(hpk, ew, ridx_h, cidx_h, init, out, ridx, cidx, hbuf, ebuf, mbuf,
             acc, hsem, esem, ssem, chained, es, ng, cpg):
    cpt = ng * cpg
    c = lax.axis_index("c")
    s = lax.axis_index("s")
    if chained:
        pltpu.sync_copy(init.at[c, s], acc.at[pl.ds(s * _RPT, _RPT)])
    else:
        pltpu.sync_copy(init, acc.at[pl.ds(s * _RPT, _RPT)])
    plsc.subcore_barrier()
    pltpu.async_copy(mbuf.at[0], out.at[c, s, pl.ds(0, _CH)], ssem)
    pltpu.async_copy(mbuf.at[1], out.at[c, s, pl.ds(0, _CH)], ssem)

    hi_mask = jnp.full((16,), -65536, jnp.int32)
    sh16 = jnp.full((16,), 16, jnp.int32)

    def _unpack_lo(v):
        return lax.bitcast_convert_type(lax.shift_left(v, sh16), jnp.float32)

    def _unpack_hi(v):
        return lax.bitcast_convert_type(lax.bitwise_and(v, hi_mask),
                                        jnp.float32)

    def _e0(g, j):
        return c * es + (s * cpt + g * cpg + j) * _CH

    def _start(g, j, slot):
        pltpu.async_copy(ew.at[pl.ds(_e0(g, j), _CH)], ebuf.at[slot], esem)
        pltpu.async_copy(hpk.at[ridx.at[j]], hbuf.at[slot], hsem)

    def _finish(g, j, slot):
        pltpu.make_async_copy(ew.at[pl.ds(_e0(g, j), _CH)], ebuf.at[slot],
                              esem).wait()
        pltpu.make_async_copy(hpk.at[ridx.at[j]], hbuf.at[slot], hsem).wait()

    def _consume(g, j, slot):
        _finish(g, j, slot)
        pltpu.make_async_copy(mbuf.at[slot], acc.at[cidx.at[j]], ssem).wait()

        def _mul_rows(h_unpack):
            def rowfn(q, carry2):
                for rr in range(4):
                    r = 4 * q + rr
                    for k in range(_H // 32):
                        lo_sl = pl.ds(k * 16, 16)
                        hi_sl = pl.ds(64 + k * 16, 16)
                        ep = ebuf[slot, r, lo_sl]
                        ha = h_unpack(hbuf[slot, r, lo_sl])
                        hb = h_unpack(hbuf[slot, r, hi_sl])
                        mbuf[slot, r, lo_sl] = ha * _unpack_lo(ep)
                        mbuf[slot, r, hi_sl] = hb * _unpack_hi(ep)
                return carry2

            lax.fori_loop(0, _CH // 4, rowfn, 0)

        @pl.when(c == 0)
        def _():
            _mul_rows(_unpack_lo)

        @pl.when(c == 1)
        def _():
            _mul_rows(_unpack_hi)

        pltpu.async_copy(mbuf.at[slot], acc.at[cidx.at[j]], ssem, add=True)

    def group(g, carry):
        pltpu.sync_copy(ridx_h.at[s, g], ridx)
        pltpu.sync_copy(cidx_h.at[s, g], cidx)
        _start(g, 0, 0)
        _start(g, 1, 1)

        def pair(p, carry1):
            j0 = 2 * p
            _consume(g, j0, 0)

            @pl.when(j0 + 2 < cpg)
            def _():
                _start(g, j0 + 2, 0)

            _consume(g, j0 + 1, 1)

            @pl.when(j0 + 3 < cpg)
            def _():
                _start(g, j0 + 3, 1)

            return carry1

        lax.fori_loop(0, cpg // 2, pair, 0)
        if cpg % 2:
            _consume(g, cpg - 1, 0)
        return carry

    lax.fori_loop(0, ng, group, 0)
    pltpu.make_async_copy(mbuf.at[0], acc.at[cidx.at[0]], ssem).wait()
    pltpu.make_async_copy(mbuf.at[1], acc.at[cidx.at[0]], ssem).wait()
    plsc.subcore_barrier()
    pltpu.sync_copy(acc.at[pl.ds(s * _RPT, _RPT)], out.at[c, s])


def _sc_scatter(h2, ew, ridx_h, cidx_h, init, chained, es, ng, cpg):
    mesh = plsc.VectorSubcoreMesh(core_axis_name="c", subcore_axis_name="s")
    run = functools.partial(
        pl.kernel,
        mesh=mesh,
        out_type=jax.ShapeDtypeStruct((2, _NS, _RPT, _H), jnp.float32),
        scratch_types=[
            pltpu.VMEM((cpg, _CH), jnp.int32),
            pltpu.VMEM((cpg, _CH), jnp.int32),
            pltpu.VMEM((2, _CH, _H), jnp.int32),
            pltpu.VMEM((2, _CH, _H // 2), jnp.int32),
            pltpu.VMEM((2, _CH, _H), jnp.float32),
            pltpu.VMEM_SHARED((_NP, _H), jnp.float32),
            pltpu.SemaphoreType.DMA,
            pltpu.SemaphoreType.DMA,
            pltpu.SemaphoreType.DMA,
        ],
    )(functools.partial(_sc_body, chained=chained, es=es, ng=ng, cpg=cpg))
    return run(h2, ew, ridx_h, cidx_h, init)


def _out_mlp_body(h, n0, n1, w3h, w3n0, w3n1, b3, w4, b4, o):
    t = (jnp.dot(h[...], w3h[...], preferred_element_type=jnp.float32)
         + jnp.dot(n0[...], w3n0[...], preferred_element_type=jnp.float32)
         + jnp.dot(n1[...], w3n1[...], preferred_element_type=jnp.float32)
         + b3[...])
    t = _silu(t)
    o[...] = h[...] + jnp.dot(t, w4[...], preferred_element_type=jnp.float32) + b4[...]


def _out_mlp(h, n0, n1, W3, b3, W4, b4):
    Nb = 2000
    full = lambda i: (0, 0)
    return pl.pallas_call(
        _out_mlp_body,
        grid=(_N // Nb,),
        in_specs=[
            pl.BlockSpec((Nb, _D), lambda i: (i, 0)),
            pl.BlockSpec((Nb, _H), lambda i: (i, 0)),
            pl.BlockSpec((Nb, _H), lambda i: (i, 0)),
            pl.BlockSpec((_D, _D), full),
            pl.BlockSpec((_H, _D), full),
            pl.BlockSpec((_H, _D), full),
            pl.BlockSpec((1, _D), full),
            pl.BlockSpec((_D, _D), full),
            pl.BlockSpec((1, _D), full),
        ],
        out_specs=pl.BlockSpec((Nb, _D), lambda i: (i, 0)),
        out_shape=jax.ShapeDtypeStruct((_N, _D), jnp.float32),
    )(h, n0, n1, W3[:_D], W3[_D:_D + _H], W3[_D + _H:],
      b3.reshape(1, _D), W4, b4.reshape(1, _D))


def kernel(h, edge_index, edge_attr, mask, W1, b1, W2, b2, W3, b3, W4, b4):
    row = edge_index[0]
    col = edge_index[1]
    hpk = _pack_bf16_pair(h[:, :_H], h[:, _H:])
    eat = edge_attr.T
    W1b = W1.astype(jnp.bfloat16)
    W2b = W2.astype(jnp.bfloat16)
    b1r = b1.reshape(1, _D)
    b2r = b2.reshape(1, _D)
    zrows = jnp.zeros((_RPT, _H), jnp.float32)
    hn4 = None
    e0 = 0
    for es, ng in _SLICES:
        cpg = es // (_NS * _CH * ng)
        ew2 = _edge_mlp(eat, W1b, b1r, W2b, b2r, e0, es).reshape(
            2 * es, _H // 2)
        ridx_h = row[e0:e0 + es].reshape(_NS, ng, cpg, _CH)
        cidx_h = col[e0:e0 + es].reshape(_NS, ng, cpg, _CH)
        init = zrows if hn4 is None else hn4
        hn4 = _sc_scatter(hpk, ew2, ridx_h, cidx_h, init,
                          chained=hn4 is not None, es=es, ng=ng, cpg=cpg)
        e0 += es
    hnf = hn4.reshape(2, _NP, _H)
    return _out_mlp(h, hnf[0], hnf[1], W3, b3, W4, b4)
